# Initial kernel scaffold; baseline (speedup 1.0000x reference)
import jax
import jax.numpy as jnp
from jax.experimental import pallas as pl


def kernel(x, edge_index, params):
    raise NotImplementedError("write your pallas kernel here")


# speedup vs baseline: 59.3849x; 59.3849x over previous
import functools

import numpy as np
import jax
import jax.numpy as jnp
from jax import lax
from jax.experimental import pallas as pl
from jax.experimental.pallas import tpu as pltpu
from jax.experimental.pallas import tpu_sc as plsc

NA = 500
N = 512
H = 64
B = 4
RT = 15
EPR = N - 1
NE = N * EPR
EPW = 16 * EPR
BN_INV = np.float32(1.0 / np.sqrt(1.0 + 1e-5))
INV_SQRT2 = np.float32(0.7071067811865476)


def _gelu(v):
    return 0.5 * v * (1.0 + lax.erf(v * INV_SQRT2))


def _dg(a, b, a_dim, b_dim):
    return lax.dot_general(a, b, (((a_dim,), (b_dim,)), ((), ())),
                           preferred_element_type=jnp.float32)


def _k1_body(x_ref, embW_ref, embb_ref,
             c0W, c0b, c0g, c0be, c1W, c1b, c1g, c1be, c2W, c2b, c2g, c2be,
             gW, gas, gad, w1a_ref, w1b_ref, b1_ref,
             h0_ref, s0_ref, d0_ref, p_ref, q_ref):
    xb = x_ref[0]
    e = _dg(xb.reshape(N * RT, 32), embW_ref[...], 1, 1) + embb_ref[...][None, :]
    e = e.reshape(N, RT, H)

    def conv(hin, W_r, b_r, g_r, be_r, d, npos, inbase):
        acc = None
        for k in range(3):
            t0 = (RT - npos) - (2 - k) * d - inbase
            sl = hin[:, t0:t0 + npos, :]
            m = _dg(sl.reshape(N * npos, H), W_r[...][:, :, k], 1, 1)
            acc = m if acc is None else acc + m
        acc = acc + b_r[...][None, :]
        acc = g_r[...][None, :] * acc * BN_INV + be_r[...][None, :]
        return _gelu(acc).reshape(N, npos, H)

    l1 = conv(e, c0W, c0b, c0g, c0be, 1, 13, 0)
    l2 = conv(l1, c1W, c1b, c1g, c1be, 2, 9, 2)
    l3 = conv(l2, c2W, c2b, c2g, c2be, 4, 1, 6)
    feat = l3[:, 0, :]

    h0 = _dg(feat, gW[...], 1, 1)
    h0_ref[...] = h0[None]
    s0_ref[...] = _dg(gas[...].reshape(1, H), h0, 1, 1)[None]
    d0_ref[...] = _dg(gad[...].reshape(1, H), h0, 1, 1)[None]
    p_ref[...] = (_dg(feat, w1a_ref[...], 1, 1) + b1_ref[...][None, :])[None]
    q_ref[...] = _dg(feat, w1b_ref[...], 1, 1)[None]


def _run_k1(xs, p):
    full = lambda a: pl.BlockSpec(a.shape, lambda b: (0,) * a.ndim)
    wargs = [p['emb_W'], p['emb_b'],
             p['conv0_W'], p['conv0_b'], p['conv0_g'], p['conv0_be'],
             p['conv1_W'], p['conv1_b'], p['conv1_g'], p['conv1_be'],
             p['conv2_W'], p['conv2_b'], p['conv2_g'], p['conv2_be'],
             p['gat0_W'], p['gat0_as'][0, 0], p['gat0_ad'][0, 0],
             p['ew_W1'][:, :H], p['ew_W1'][:, H:], p['ew_b1']]
    return pl.pallas_call(
        _k1_body,
        grid=(B,),
        in_specs=[pl.BlockSpec((1, N, RT, 32), lambda b: (b, 0, 0, 0))] +
                 [full(a) for a in wargs],
        out_specs=[pl.BlockSpec((1, N, H), lambda b: (b, 0, 0)),
                   pl.BlockSpec((1, 1, N), lambda b: (b, 0, 0)),
                   pl.BlockSpec((1, 1, N), lambda b: (b, 0, 0)),
                   pl.BlockSpec((1, N, H), lambda b: (b, 0, 0)),
                   pl.BlockSpec((1, N, H), lambda b: (b, 0, 0))],
        out_shape=[jax.ShapeDtypeStruct((B, N, H), jnp.float32),
                   jax.ShapeDtypeStruct((B, 1, N), jnp.float32),
                   jax.ShapeDtypeStruct((B, 1, N), jnp.float32),
                   jax.ShapeDtypeStruct((B, N, H), jnp.float32),
                   jax.ShapeDtypeStruct((B, N, H), jnp.float32)],
    )(xs, *wargs)


IB = 32


def _k2_body(p_ref, q_ref, w2_ref, b2_ref, ew_ref):
    pb = p_ref[0]
    qb = q_ref[0]
    t = jnp.maximum(pb[:, None, :] + qb[None, :, :], 0.0)
    z = _dg(t, w2_ref[...], 2, 0) + b2_ref[0, 0]
    ew_ref[...] = jax.nn.sigmoid(z)[None]


def _run_k2(P, Q, p):
    w2 = p['ew_W2'][0]
    b2 = p['ew_b2'].reshape(1, 1)
    return pl.pallas_call(
        _k2_body,
        grid=(B, N // IB),
        in_specs=[pl.BlockSpec((1, IB, H), lambda b, i: (b, i, 0)),
                  pl.BlockSpec((1, N, H), lambda b, i: (b, 0, 0)),
                  pl.BlockSpec((H,), lambda b, i: (0,)),
                  pl.BlockSpec((1, 1), lambda b, i: (0, 0))],
        out_specs=pl.BlockSpec((1, IB, N), lambda b, i: (b, i, 0)),
        out_shape=jax.ShapeDtypeStruct((B, N, N), jnp.float32),
    )(P, Q, w2, b2)


def _sc_body(s_hbm, d_hbm, ew_hbm, src_hbm, dst_hbm, out_hbm,
             src_c, dst_c, s_all, d_all, ew_r, p_buf, idx_buf, zbuf, apre):
    cid = lax.axis_index("c")
    sid = lax.axis_index("s")
    wid = cid * 16 + sid

    pltpu.sync_copy(s_hbm, s_all)
    pltpu.sync_copy(d_hbm, d_all)
    pltpu.sync_copy(src_hbm.at[pl.ds(wid * EPW, EPW)], src_c.at[pl.ds(0, EPW)])
    pltpu.sync_copy(dst_hbm.at[pl.ds(wid * EPW, EPW)], dst_c.at[pl.ds(0, EPW)])
    src_c[pl.ds(EPW, 16)] = jnp.zeros((16,), jnp.int32)
    dst_c[pl.ds(EPW, 16)] = jnp.zeros((16,), jnp.int32)

    def _z(j, _):
        zbuf[pl.ds(j * 16, 16)] = jnp.zeros((16,), jnp.float32)
        return _
    lax.fori_loop(0, 128, _z, None)

    def _za(k, _):
        pltpu.sync_copy(zbuf, apre.at[pl.ds(sid * 65536 + k * 2048, 2048)])
        return _
    lax.fori_loop(0, 32, _za, None)
    plsc.subcore_barrier()

    iota = lax.iota(jnp.int32, 16)

    def batch_body(b, _):
        pltpu.sync_copy(ew_hbm.at[b, pl.ds(wid * 16, 16)], ew_r)

        def step(k, _c):
            i_loc = k // 32
            t = k % 32
            i_glob = wid * 16 + i_loc
            j = t * 16 + iota
            valid = j < EPR
            jfull = jnp.where(valid, j + jnp.where(j >= i_glob, 1, 0), 0)
            e_loc = i_loc * EPR + j
            srcv = plsc.load_gather(src_c, [e_loc])
            dstv = plsc.load_gather(dst_c, [e_loc])
            sg = plsc.load_gather(s_all, [b * N + srcv])
            dg = plsc.load_gather(d_all, [b * N + dstv])
            tt = sg + dg
            r = jnp.where(tt > 0, tt, 0.2 * tt)
            ewv = plsc.load_gather(ew_r, [jnp.full((16,), i_loc, jnp.int32), jfull])
            pv = jnp.where(valid, jnp.exp(r * ewv), 0.0)
            fidx = b * (N * N) + dstv * N + srcv
            row = i_loc * 4 + t // 8
            col = (t % 8) * 16
            p_buf[row, pl.ds(col, 16)] = pv
            idx_buf[row, pl.ds(col, 16)] = fidx
            return _c
        lax.fori_loop(0, 512, step, None)

        def scat(jr, _c):
            pltpu.sync_copy(p_buf.at[jr], apre.at[idx_buf.at[jr]], add=True)
            return _c
        lax.fori_loop(0, 64, scat, None)
        return _
    lax.fori_loop(0, B, batch_body, None)

    plsc.subcore_barrier()
    pltpu.sync_copy(apre.at[pl.ds(sid * 65536, 65536)],
                    out_hbm.at[cid, pl.ds(sid * 65536, 65536)])


def _run_sc(s, d, EW, src, dst):
    mesh = plsc.VectorSubcoreMesh(core_axis_name="c", subcore_axis_name="s")
    f = functools.partial(
        pl.kernel, _sc_body, mesh=mesh,
        compiler_params=pltpu.CompilerParams(needs_layout_passes=False),
        out_type=jax.ShapeDtypeStruct((2, B * N * N), jnp.float32),
        scratch_types=[
            pltpu.VMEM((EPW + 16,), jnp.int32),
            pltpu.VMEM((EPW + 16,), jnp.int32),
            pltpu.VMEM((B * N,), jnp.float32),
            pltpu.VMEM((B * N,), jnp.float32),
            pltpu.VMEM((16, N), jnp.float32),
            pltpu.VMEM((64, 128), jnp.float32),
            pltpu.VMEM((64, 128), jnp.int32),
            pltpu.VMEM((2048,), jnp.float32),
            pltpu.VMEM_SHARED((N * N * B,), jnp.float32),
        ],
    )()
    return f(s, d, EW, src, dst)


def _k5_body(a0_ref, a1_ref, h_ref, gW, gas, gad, h1_ref, s1_ref, d1_ref):
    A = a0_ref[0] + a1_ref[0]
    rs = jnp.sum(A, axis=1, keepdims=True)
    feat = _dg(A, h_ref[0], 1, 0) / (rs + 1e-8)
    h1 = _dg(feat, gW[...], 1, 1)
    h1_ref[...] = h1[None]
    s1_ref[...] = _dg(gas[...].reshape(1, H), h1, 1, 1)[None]
    d1_ref[...] = _dg(gad[...].reshape(1, H), h1, 1, 1)[None]


def _run_k5(A0, A1, h, W, a_s, a_d):
    full = lambda a: pl.BlockSpec(a.shape, lambda b: (0,) * a.ndim)
    return pl.pallas_call(
        _k5_body,
        grid=(B,),
        in_specs=[pl.BlockSpec((1, N, N), lambda b: (b, 0, 0)),
                  pl.BlockSpec((1, N, N), lambda b: (b, 0, 0)),
                  pl.BlockSpec((1, N, H), lambda b: (b, 0, 0)),
                  full(W), full(a_s), full(a_d)],
        out_specs=[pl.BlockSpec((1, N, H), lambda b: (b, 0, 0)),
                   pl.BlockSpec((1, 1, N), lambda b: (b, 0, 0)),
                   pl.BlockSpec((1, 1, N), lambda b: (b, 0, 0))],
        out_shape=[jax.ShapeDtypeStruct((B, N, H), jnp.float32),
                   jax.ShapeDtypeStruct((B, 1, N), jnp.float32),
                   jax.ShapeDtypeStruct((B, 1, N), jnp.float32)],
    )(A0, A1, h, W, a_s, a_d)


def _k6_body(a0_ref, a1_ref, h_ref, w1_ref, b1_ref, w2_ref, b2_ref, out_ref):
    A = a0_ref[0] + a1_ref[0]
    rs = jnp.sum(A, axis=1, keepdims=True)
    feat = _dg(A, h_ref[0], 1, 0) / (rs + 1e-8)
    t1 = jnp.concatenate(
        [jnp.sum(feat * w1_ref[o], axis=1, keepdims=True) for o in range(32)],
        axis=1) + b1_ref[...]
    t1 = jnp.maximum(t1, 0.0)
    cols = [jnp.sum(t1 * w2_ref[o], axis=1, keepdims=True) for o in range(3)]
    cols.append(jnp.zeros((N, 5), jnp.float32))
    pred = jnp.concatenate(cols, axis=1) + b2_ref[...]
    out_ref[...] = pred[None]


def _run_k6(A0, A1, h, W1p, b1p, W2p, b2p):
    full = lambda a: pl.BlockSpec(a.shape, lambda b: (0,) * a.ndim)
    return pl.pallas_call(
        _k6_body,
        grid=(B,),
        in_specs=[pl.BlockSpec((1, N, N), lambda b: (b, 0, 0)),
                  pl.BlockSpec((1, N, N), lambda b: (b, 0, 0)),
                  pl.BlockSpec((1, N, H), lambda b: (b, 0, 0)),
                  full(W1p), full(b1p), full(W2p), full(b2p)],
        out_specs=pl.BlockSpec((1, N, 8), lambda b: (b, 0, 0)),
        out_shape=jax.ShapeDtypeStruct((B, N, 8), jnp.float32),
    )(A0, A1, h, W1p, b1p, W2p, b2p)


def kernel(x, edge_index, params):
    xs = x[:, :, 49:, :]
    h0, s0, d0, P, Q = _run_k1(xs, params)
    EW = _run_k2(P, Q, params)
    src = edge_index[0]
    dst = edge_index[1]

    W1p = jnp.pad(params['head_W1'].transpose(1, 0, 2), ((0, 0), (0, N - NA), (0, 0)))
    b1p = jnp.pad(params['head_b1'], ((0, N - NA), (0, 0)))
    W2p = jnp.pad(params['head_W2'].transpose(1, 0, 2), ((0, 0), (0, N - NA), (0, 0)))
    b2p = jnp.pad(params['head_b2'], ((0, N - NA), (0, 5)))

    h, s, d = h0, s0, d0
    for l in range(3):
        parts = _run_sc(s.reshape(-1), d.reshape(-1), EW, src, dst)
        A0 = parts[0].reshape(B, N, N)
        A1 = parts[1].reshape(B, N, N)
        if l < 2:
            nm = 'gat%d' % (l + 1)
            h, s, d = _run_k5(A0, A1, h, params[nm + '_W'],
                              params[nm + '_as'][0, 0], params[nm + '_ad'][0, 0])
        else:
            pred = _run_k6(A0, A1, h, W1p, b1p, W2p, b2p)
    return pred[:, :NA, :3]


# speedup vs baseline: 122.3560x; 2.0604x over previous
import functools

import numpy as np
import jax
import jax.numpy as jnp
from jax import lax
from jax.experimental import pallas as pl
from jax.experimental.pallas import tpu as pltpu
from jax.experimental.pallas import tpu_sc as plsc

NA = 500
N = 512
H = 64
B = 4
RT = 15
EPR = N - 1
NE = N * EPR
EPW = 16 * EPR
BN_INV = np.float32(1.0 / np.sqrt(1.0 + 1e-5))
INV_SQRT2 = np.float32(0.7071067811865476)


def _gelu(v):
    return 0.5 * v * (1.0 + lax.erf(v * INV_SQRT2))


def _dg(a, b, a_dim, b_dim):
    return lax.dot_general(a, b, (((a_dim,), (b_dim,)), ((), ())),
                           preferred_element_type=jnp.float32)


def _k1_body(x_ref, embW_ref, embb_ref,
             c0W, c0b, c0g, c0be, c1W, c1b, c1g, c1be, c2W, c2b, c2g, c2be,
             gW, gas, gad, w1a_ref, w1b_ref, b1_ref,
             h0_ref, s0_ref, d0_ref, p_ref, q_ref):
    xb = x_ref[0]
    e = _dg(xb.reshape(N * RT, 32), embW_ref[...], 1, 1) + embb_ref[...][None, :]
    e = e.reshape(N, RT, H)

    def conv(hin, W_r, b_r, g_r, be_r, d, npos, inbase):
        acc = None
        for k in range(3):
            t0 = (RT - npos) - (2 - k) * d - inbase
            sl = hin[:, t0:t0 + npos, :]
            m = _dg(sl.reshape(N * npos, H), W_r[...][:, :, k], 1, 1)
            acc = m if acc is None else acc + m
        acc = acc + b_r[...][None, :]
        acc = g_r[...][None, :] * acc * BN_INV + be_r[...][None, :]
        return _gelu(acc).reshape(N, npos, H)

    l1 = conv(e, c0W, c0b, c0g, c0be, 1, 13, 0)
    l2 = conv(l1, c1W, c1b, c1g, c1be, 2, 9, 2)
    l3 = conv(l2, c2W, c2b, c2g, c2be, 4, 1, 6)
    feat = l3[:, 0, :]

    h0 = _dg(feat, gW[...], 1, 1)
    h0_ref[...] = h0[None]
    s0_ref[...] = _dg(gas[...].reshape(1, H), h0, 1, 1)[None]
    d0_ref[...] = _dg(gad[...].reshape(1, H), h0, 1, 1)[None]
    p_ref[...] = (_dg(feat, w1a_ref[...], 1, 1) + b1_ref[...][None, :])[None]
    q_ref[...] = _dg(w1b_ref[...], feat, 1, 1)[None]


def _run_k1(xs, p):
    full = lambda a: pl.BlockSpec(a.shape, lambda b: (0,) * a.ndim)
    wargs = [p['emb_W'], p['emb_b'],
             p['conv0_W'], p['conv0_b'], p['conv0_g'], p['conv0_be'],
             p['conv1_W'], p['conv1_b'], p['conv1_g'], p['conv1_be'],
             p['conv2_W'], p['conv2_b'], p['conv2_g'], p['conv2_be'],
             p['gat0_W'], p['gat0_as'][0, 0], p['gat0_ad'][0, 0],
             p['ew_W1'][:, :H], p['ew_W1'][:, H:], p['ew_b1']]
    return pl.pallas_call(
        _k1_body,
        grid=(B,),
        in_specs=[pl.BlockSpec((1, N, RT, 32), lambda b: (b, 0, 0, 0))] +
                 [full(a) for a in wargs],
        out_specs=[pl.BlockSpec((1, N, H), lambda b: (b, 0, 0)),
                   pl.BlockSpec((1, 1, N), lambda b: (b, 0, 0)),
                   pl.BlockSpec((1, 1, N), lambda b: (b, 0, 0)),
                   pl.BlockSpec((1, N, H), lambda b: (b, 0, 0)),
                   pl.BlockSpec((1, H, N), lambda b: (b, 0, 0))],
        out_shape=[jax.ShapeDtypeStruct((B, N, H), jnp.float32),
                   jax.ShapeDtypeStruct((B, 1, N), jnp.float32),
                   jax.ShapeDtypeStruct((B, 1, N), jnp.float32),
                   jax.ShapeDtypeStruct((B, N, H), jnp.float32),
                   jax.ShapeDtypeStruct((B, H, N), jnp.float32)],
    )(xs, *wargs)


IB = 32


def _k2_body(p_ref, q_ref, w2_ref, b2_ref, ew_ref):
    pbT = jnp.transpose(p_ref[0], (1, 0))
    qt = q_ref[0]
    w2 = w2_ref[...]
    b2 = b2_ref[0, 0]
    for i in range(IB):
        t = jnp.maximum(pbT[:, i:i + 1] + qt, 0.0)
        z = jnp.sum(t * w2, axis=0, keepdims=True) + b2
        ew_ref[0, i:i + 1, :] = jax.nn.sigmoid(z)


def _run_k2(P, Q, p):
    w2 = p['ew_W2'].reshape(H, 1)
    b2 = p['ew_b2'].reshape(1, 1)
    return pl.pallas_call(
        _k2_body,
        grid=(B, N // IB),
        in_specs=[pl.BlockSpec((1, IB, H), lambda b, i: (b, i, 0)),
                  pl.BlockSpec((1, H, N), lambda b, i: (b, 0, 0)),
                  pl.BlockSpec((H, 1), lambda b, i: (0, 0)),
                  pl.BlockSpec((1, 1), lambda b, i: (0, 0))],
        out_specs=pl.BlockSpec((1, IB, N), lambda b, i: (b, i, 0)),
        out_shape=jax.ShapeDtypeStruct((B, N, N), jnp.float32),
    )(P, Q, w2, b2)


def _sc_body(s_hbm, d_hbm, ew_hbm, src_hbm, dst_hbm, out_hbm,
             src_c, dst_c, s_all, d_all, ew_r, p_buf, idx_buf, zbuf, apre):
    cid = lax.axis_index("c")
    sid = lax.axis_index("s")
    wid = cid * 16 + sid

    pltpu.sync_copy(s_hbm, s_all)
    pltpu.sync_copy(d_hbm, d_all)
    pltpu.sync_copy(src_hbm.at[pl.ds(wid * EPW, EPW)], src_c.at[pl.ds(0, EPW)])
    pltpu.sync_copy(dst_hbm.at[pl.ds(wid * EPW, EPW)], dst_c.at[pl.ds(0, EPW)])
    src_c[pl.ds(EPW, 16)] = jnp.zeros((16,), jnp.int32)
    dst_c[pl.ds(EPW, 16)] = jnp.zeros((16,), jnp.int32)

    def _z(j, _):
        zbuf[pl.ds(j * 16, 16)] = jnp.zeros((16,), jnp.float32)
        return _
    lax.fori_loop(0, 128, _z, None)

    iota = lax.iota(jnp.int32, 16)
    HALF = 2 * N * N
    TSL = HALF // 16

    for ph in range(2):
        def _za(k, _):
            pltpu.sync_copy(zbuf, apre.at[pl.ds(sid * TSL + k * 2048, 2048)])
            return _
        lax.fori_loop(0, TSL // 2048, _za, None)
        for b in range(2):
            pltpu.sync_copy(ew_hbm.at[2 * ph + b, pl.ds(wid * 16, 16)],
                            ew_r.at[pl.ds(b * 16, 16)])
        plsc.subcore_barrier()

        @plsc.parallel_loop(0, 512, unroll=2)
        def step(k):
            i_loc = k // 32
            t = k % 32
            i_glob = wid * 16 + i_loc
            j = t * 16 + iota
            valid = j < EPR
            jfull = jnp.where(valid, j + jnp.where(j >= i_glob, 1, 0), 0)
            e_loc = i_loc * EPR + j
            srcv = plsc.load_gather(src_c, [e_loc])
            dstv = plsc.load_gather(dst_c, [e_loc])
            base = dstv * N + srcv
            row0 = i_loc * 4 + t // 8
            col = (t % 8) * 16
            for b in range(2):
                gb = 2 * ph + b
                sg = plsc.load_gather(s_all, [gb * N + srcv])
                dg = plsc.load_gather(d_all, [gb * N + dstv])
                tt = sg + dg
                r = jnp.where(tt > 0, tt, 0.2 * tt)
                ewv = plsc.load_gather(
                    ew_r, [jnp.full((16,), b * 16 + i_loc, jnp.int32), jfull])
                pv = jnp.where(valid, jnp.exp(r * ewv), 0.0)
                p_buf[b * 64 + row0, pl.ds(col, 16)] = pv
                idx_buf[b * 64 + row0, pl.ds(col, 16)] = base + b * (N * N)

        def scat(jr, _c):
            pltpu.sync_copy(p_buf.at[jr], apre.at[idx_buf.at[jr]], add=True)
            return _c
        lax.fori_loop(0, 128, scat, None)
        plsc.subcore_barrier()
        pltpu.sync_copy(
            apre.at[pl.ds(sid * TSL, TSL)],
            out_hbm.at[cid, pl.ds(ph * HALF + sid * TSL, TSL)])


def _run_sc(s, d, EW, src, dst):
    mesh = plsc.VectorSubcoreMesh(core_axis_name="c", subcore_axis_name="s")
    f = functools.partial(
        pl.kernel, _sc_body, mesh=mesh,
        compiler_params=pltpu.CompilerParams(needs_layout_passes=False),
        out_type=jax.ShapeDtypeStruct((2, B * N * N), jnp.float32),
        scratch_types=[
            pltpu.VMEM((EPW + 16,), jnp.int32),
            pltpu.VMEM((EPW + 16,), jnp.int32),
            pltpu.VMEM((B * N,), jnp.float32),
            pltpu.VMEM((B * N,), jnp.float32),
            pltpu.VMEM((32, N), jnp.float32),
            pltpu.VMEM((128, 128), jnp.float32),
            pltpu.VMEM((128, 128), jnp.int32),
            pltpu.VMEM((2048,), jnp.float32),
            pltpu.VMEM_SHARED((2 * N * N,), jnp.float32),
        ],
    )()
    return f(s, d, EW, src, dst)


def _k5_body(a0_ref, a1_ref, h_ref, gW, gas, gad, h1_ref, s1_ref, d1_ref):
    A = a0_ref[0, 0] + a1_ref[0, 0]
    rs = jnp.sum(A, axis=1, keepdims=True)
    feat = _dg(A, h_ref[0], 1, 0) / (rs + 1e-8)
    h1 = _dg(feat, gW[...], 1, 1)
    h1_ref[...] = h1[None]
    s1_ref[...] = _dg(gas[...].reshape(1, H), h1, 1, 1)[None]
    d1_ref[...] = _dg(gad[...].reshape(1, H), h1, 1, 1)[None]


def _run_k5(A, h, W, a_s, a_d):
    full = lambda a: pl.BlockSpec(a.shape, lambda b: (0,) * a.ndim)
    return pl.pallas_call(
        _k5_body,
        grid=(B,),
        in_specs=[pl.BlockSpec((1, 1, N, N), lambda b: (0, b, 0, 0)),
                  pl.BlockSpec((1, 1, N, N), lambda b: (1, b, 0, 0)),
                  pl.BlockSpec((1, N, H), lambda b: (b, 0, 0)),
                  full(W), full(a_s), full(a_d)],
        out_specs=[pl.BlockSpec((1, N, H), lambda b: (b, 0, 0)),
                   pl.BlockSpec((1, 1, N), lambda b: (b, 0, 0)),
                   pl.BlockSpec((1, 1, N), lambda b: (b, 0, 0))],
        out_shape=[jax.ShapeDtypeStruct((B, N, H), jnp.float32),
                   jax.ShapeDtypeStruct((B, 1, N), jnp.float32),
                   jax.ShapeDtypeStruct((B, 1, N), jnp.float32)],
    )(A, A, h, W, a_s, a_d)


def _k6_body(a0_ref, a1_ref, h_ref, w1_ref, b1_ref, w2_ref, b2_ref, out_ref):
    A = a0_ref[0, 0] + a1_ref[0, 0]
    rs = jnp.sum(A, axis=1, keepdims=True)
    feat = _dg(A, h_ref[0], 1, 0) / (rs + 1e-8)
    af = feat[:NA]
    t1 = jnp.concatenate(
        [jnp.sum(af * w1_ref[:, o, :], axis=1, keepdims=True) for o in range(32)],
        axis=1) + b1_ref[...]
    t1 = jnp.maximum(t1, 0.0)
    cols = [jnp.sum(t1 * w2_ref[:, o, :], axis=1, keepdims=True) for o in range(3)]
    pred = jnp.concatenate(cols, axis=1) + b2_ref[...]
    out_ref[...] = pred[None]


def _run_k6(A, h, W1, b1, W2, b2):
    full = lambda a: pl.BlockSpec(a.shape, lambda b: (0,) * a.ndim)
    return pl.pallas_call(
        _k6_body,
        grid=(B,),
        in_specs=[pl.BlockSpec((1, 1, N, N), lambda b: (0, b, 0, 0)),
                  pl.BlockSpec((1, 1, N, N), lambda b: (1, b, 0, 0)),
                  pl.BlockSpec((1, N, H), lambda b: (b, 0, 0)),
                  full(W1), full(b1), full(W2), full(b2)],
        out_specs=pl.BlockSpec((1, NA, 3), lambda b: (b, 0, 0)),
        out_shape=jax.ShapeDtypeStruct((B, NA, 3), jnp.float32),
    )(A, A, h, W1, b1, W2, b2)


def kernel(x, edge_index, params):
    xs = x[:, :, 49:, :]
    h0, s0, d0, P, Qt = _run_k1(xs, params)
    EW = _run_k2(P, Qt, params)
    src = edge_index[0]
    dst = edge_index[1]

    h, s, d = h0, s0, d0
    for l in range(3):
        parts = _run_sc(s.reshape(-1), d.reshape(-1), EW, src, dst)
        A = parts.reshape(2, B, N, N)
        if l < 2:
            nm = 'gat%d' % (l + 1)
            h, s, d = _run_k5(A, h, params[nm + '_W'],
                              params[nm + '_as'][0, 0], params[nm + '_ad'][0, 0])
        else:
            pred = _run_k6(A, h, params['head_W1'], params['head_b1'],
                           params['head_W2'], params['head_b2'])
    return pred


# speedup vs baseline: 145.9124x; 1.1925x over previous
import functools

import numpy as np
import jax
import jax.numpy as jnp
from jax import lax
from jax.experimental import pallas as pl
from jax.experimental.pallas import tpu as pltpu
from jax.experimental.pallas import tpu_sc as plsc

NA = 500
N = 512
H = 64
B = 4
RT = 15
EPR = N - 1
NE = N * EPR
EPW = 16 * EPR
BN_INV = np.float32(1.0 / np.sqrt(1.0 + 1e-5))
INV_SQRT2 = np.float32(0.7071067811865476)


def _gelu(v):
    return 0.5 * v * (1.0 + lax.erf(v * INV_SQRT2))


def _dg(a, b, a_dim, b_dim):
    return lax.dot_general(a, b, (((a_dim,), (b_dim,)), ((), ())),
                           preferred_element_type=jnp.float32)


def _k1_body(x_ref, embW_ref, embb_ref,
             c0W, c0b, c0g, c0be, c1W, c1b, c1g, c1be, c2W, c2b, c2g, c2be,
             gW, gas, gad, w1a_ref, w1b_ref, b1_ref,
             h0_ref, s0_ref, d0_ref, p_ref, q_ref):
    xb = x_ref[0]
    e = _dg(xb.reshape(RT * N, 32), embW_ref[...], 1, 1) + embb_ref[...][None, :]
    e = e.reshape(RT, N, H)

    def conv(hin, W_r, b_r, g_r, be_r, d, npos, inbase):
        acc = None
        for k in range(3):
            t0 = (RT - npos) - (2 - k) * d - inbase
            sl = hin[t0:t0 + npos]
            m = _dg(sl.reshape(npos * N, H), W_r[...][:, :, k], 1, 1)
            acc = m if acc is None else acc + m
        acc = acc + b_r[...][None, :]
        acc = g_r[...][None, :] * acc * BN_INV + be_r[...][None, :]
        return _gelu(acc).reshape(npos, N, H)

    l1 = conv(e, c0W, c0b, c0g, c0be, 1, 13, 0)
    l2 = conv(l1, c1W, c1b, c1g, c1be, 2, 9, 2)
    l3 = conv(l2, c2W, c2b, c2g, c2be, 4, 1, 6)
    feat = l3[0]

    h0 = _dg(feat, gW[...], 1, 1)
    h0_ref[...] = h0[None]
    s0_ref[...] = _dg(gas[...].reshape(1, H), h0, 1, 1)[None]
    d0_ref[...] = _dg(gad[...].reshape(1, H), h0, 1, 1)[None]
    p_ref[...] = (_dg(feat, w1a_ref[...], 1, 1) + b1_ref[...][None, :])[None]
    q_ref[...] = _dg(w1b_ref[...], feat, 1, 1)[None]


def _run_k1(xs, p):
    full = lambda a: pl.BlockSpec(a.shape, lambda b: (0,) * a.ndim)
    wargs = [p['emb_W'], p['emb_b'],
             p['conv0_W'], p['conv0_b'], p['conv0_g'], p['conv0_be'],
             p['conv1_W'], p['conv1_b'], p['conv1_g'], p['conv1_be'],
             p['conv2_W'], p['conv2_b'], p['conv2_g'], p['conv2_be'],
             p['gat0_W'], p['gat0_as'][0, 0], p['gat0_ad'][0, 0],
             p['ew_W1'][:, :H], p['ew_W1'][:, H:], p['ew_b1']]
    return pl.pallas_call(
        _k1_body,
        grid=(B,),
        in_specs=[pl.BlockSpec((1, RT, N, 32), lambda b: (b, 0, 0, 0))] +
                 [full(a) for a in wargs],
        out_specs=[pl.BlockSpec((1, N, H), lambda b: (b, 0, 0)),
                   pl.BlockSpec((1, 1, N), lambda b: (b, 0, 0)),
                   pl.BlockSpec((1, 1, N), lambda b: (b, 0, 0)),
                   pl.BlockSpec((1, N, H), lambda b: (b, 0, 0)),
                   pl.BlockSpec((1, H, N), lambda b: (b, 0, 0))],
        out_shape=[jax.ShapeDtypeStruct((B, N, H), jnp.float32),
                   jax.ShapeDtypeStruct((B, 1, N), jnp.float32),
                   jax.ShapeDtypeStruct((B, 1, N), jnp.float32),
                   jax.ShapeDtypeStruct((B, N, H), jnp.float32),
                   jax.ShapeDtypeStruct((B, H, N), jnp.float32)],
    )(xs, *wargs)


IB = 32


def _k2_body(p_ref, q_ref, w2_ref, b2_ref, ew_ref):
    pbT = jnp.transpose(p_ref[0], (1, 0))
    qt = q_ref[0]
    w2 = w2_ref[...]
    b2 = b2_ref[0, 0]
    for i in range(IB):
        t = jnp.maximum(pbT[:, i:i + 1] + qt, 0.0)
        z = jnp.sum(t * w2, axis=0, keepdims=True) + b2
        ew_ref[0, i:i + 1, :] = jax.nn.sigmoid(z)


def _run_k2(P, Q, p):
    w2 = p['ew_W2'].reshape(H, 1)
    b2 = p['ew_b2'].reshape(1, 1)
    return pl.pallas_call(
        _k2_body,
        grid=(B, N // IB),
        in_specs=[pl.BlockSpec((1, IB, H), lambda b, i: (b, i, 0)),
                  pl.BlockSpec((1, H, N), lambda b, i: (b, 0, 0)),
                  pl.BlockSpec((H, 1), lambda b, i: (0, 0)),
                  pl.BlockSpec((1, 1), lambda b, i: (0, 0))],
        out_specs=pl.BlockSpec((1, IB, N), lambda b, i: (b, i, 0)),
        out_shape=jax.ShapeDtypeStruct((B, N, N), jnp.float32),
    )(P, Q, w2, b2)


def _sc_body(s_hbm, d_hbm, ew_hbm, src_hbm, dst_hbm, out_hbm,
             src_c, dst_c, s_all, d_all, ew_r, p_buf, idx_buf, zbuf, apre):
    cid = lax.axis_index("c")
    sid = lax.axis_index("s")
    wid = cid * 16 + sid

    pltpu.sync_copy(s_hbm, s_all)
    pltpu.sync_copy(d_hbm, d_all)
    pltpu.sync_copy(src_hbm.at[pl.ds(wid * EPW, EPW)], src_c.at[pl.ds(0, EPW)])
    pltpu.sync_copy(dst_hbm.at[pl.ds(wid * EPW, EPW)], dst_c.at[pl.ds(0, EPW)])
    src_c[pl.ds(EPW, 16)] = jnp.zeros((16,), jnp.int32)
    dst_c[pl.ds(EPW, 16)] = jnp.zeros((16,), jnp.int32)

    def _z(j, _):
        zbuf[pl.ds(j * 16, 16)] = jnp.zeros((16,), jnp.float32)
        return _
    lax.fori_loop(0, 128, _z, None)

    iota = lax.iota(jnp.int32, 16)
    HALF = 2 * N * N
    TSL = HALF // 16

    for ph in range(2):
        def _za(k, _):
            pltpu.sync_copy(zbuf, apre.at[pl.ds(sid * TSL + k * 2048, 2048)])
            return _
        lax.fori_loop(0, TSL // 2048, _za, None)
        for b in range(2):
            pltpu.sync_copy(ew_hbm.at[2 * ph + b, pl.ds(wid * 16, 16)],
                            ew_r.at[pl.ds(b * 16, 16)])
        plsc.subcore_barrier()

        for i_loc in range(16):
            i_glob = wid * 16 + i_loc
            e_base = i_loc * EPR
            row_base = i_loc * 4

            @plsc.parallel_loop(0, 32, unroll=4)
            def step(t):
                j = t * 16 + iota
                valid = j < EPR
                jfull = jnp.where(valid, j + jnp.where(j >= i_glob, 1, 0), 0)
                e_loc = e_base + j
                srcv = plsc.load_gather(src_c, [e_loc])
                dstv = plsc.load_gather(dst_c, [e_loc])
                base = dstv * N + srcv
                row0 = row_base + t // 8
                col = (t % 8) * 16
                for b in range(2):
                    gb = 2 * ph + b
                    sg = plsc.load_gather(s_all, [gb * N + srcv])
                    dg = plsc.load_gather(d_all, [gb * N + dstv])
                    tt = sg + dg
                    r = jnp.where(tt > 0, tt, 0.2 * tt)
                    ewv = plsc.load_gather(
                        ew_r, [jnp.full((16,), b * 16 + i_loc, jnp.int32), jfull])
                    pv = jnp.where(valid, jnp.exp(r * ewv), 0.0)
                    p_buf[b * 64 + row0, pl.ds(col, 16)] = pv
                    idx_buf[b * 64 + row0, pl.ds(col, 16)] = base + b * (N * N)

        def scat(jr, _c):
            pltpu.sync_copy(p_buf.at[jr], apre.at[idx_buf.at[jr]], add=True)
            return _c
        lax.fori_loop(0, 128, scat, None)
        plsc.subcore_barrier()
        pltpu.sync_copy(
            apre.at[pl.ds(sid * TSL, TSL)],
            out_hbm.at[cid, pl.ds(ph * HALF + sid * TSL, TSL)])


def _run_sc(s, d, EW, src, dst):
    mesh = plsc.VectorSubcoreMesh(core_axis_name="c", subcore_axis_name="s")
    f = functools.partial(
        pl.kernel, _sc_body, mesh=mesh,
        compiler_params=pltpu.CompilerParams(needs_layout_passes=False),
        out_type=jax.ShapeDtypeStruct((2, B * N * N), jnp.float32),
        scratch_types=[
            pltpu.VMEM((EPW + 16,), jnp.int32),
            pltpu.VMEM((EPW + 16,), jnp.int32),
            pltpu.VMEM((B * N,), jnp.float32),
            pltpu.VMEM((B * N,), jnp.float32),
            pltpu.VMEM((32, N), jnp.float32),
            pltpu.VMEM((128, 128), jnp.float32),
            pltpu.VMEM((128, 128), jnp.int32),
            pltpu.VMEM((2048,), jnp.float32),
            pltpu.VMEM_SHARED((2 * N * N,), jnp.float32),
        ],
    )()
    return f(s, d, EW, src, dst)


def _k5_body(a0_ref, a1_ref, h_ref, gW, gas, gad, h1_ref, s1_ref, d1_ref):
    A = a0_ref[0, 0] + a1_ref[0, 0]
    rs = jnp.sum(A, axis=1, keepdims=True)
    feat = _dg(A, h_ref[0], 1, 0) / (rs + 1e-8)
    h1 = _dg(feat, gW[...], 1, 1)
    h1_ref[...] = h1[None]
    s1_ref[...] = _dg(gas[...].reshape(1, H), h1, 1, 1)[None]
    d1_ref[...] = _dg(gad[...].reshape(1, H), h1, 1, 1)[None]


def _run_k5(A, h, W, a_s, a_d):
    full = lambda a: pl.BlockSpec(a.shape, lambda b: (0,) * a.ndim)
    return pl.pallas_call(
        _k5_body,
        grid=(B,),
        in_specs=[pl.BlockSpec((1, 1, N, N), lambda b: (0, b, 0, 0)),
                  pl.BlockSpec((1, 1, N, N), lambda b: (1, b, 0, 0)),
                  pl.BlockSpec((1, N, H), lambda b: (b, 0, 0)),
                  full(W), full(a_s), full(a_d)],
        out_specs=[pl.BlockSpec((1, N, H), lambda b: (b, 0, 0)),
                   pl.BlockSpec((1, 1, N), lambda b: (b, 0, 0)),
                   pl.BlockSpec((1, 1, N), lambda b: (b, 0, 0))],
        out_shape=[jax.ShapeDtypeStruct((B, N, H), jnp.float32),
                   jax.ShapeDtypeStruct((B, 1, N), jnp.float32),
                   jax.ShapeDtypeStruct((B, 1, N), jnp.float32)],
    )(A, A, h, W, a_s, a_d)


def _k6_body(a0_ref, a1_ref, h_ref, w1_ref, b1_ref, w2_ref, b2_ref, out_ref):
    A = a0_ref[0, 0] + a1_ref[0, 0]
    rs = jnp.sum(A, axis=1, keepdims=True)
    feat = _dg(A, h_ref[0], 1, 0) / (rs + 1e-8)
    afT = jnp.transpose(feat[:NA], (1, 0))
    t1 = jnp.concatenate(
        [jnp.sum(afT * w1_ref[o], axis=0, keepdims=True) for o in range(32)],
        axis=0) + b1_ref[...]
    t1 = jnp.maximum(t1, 0.0)
    pred = jnp.concatenate(
        [jnp.sum(t1 * w2_ref[o], axis=0, keepdims=True) for o in range(3)],
        axis=0) + b2_ref[...]
    out_ref[...] = pred[None]


def _run_k6(A, h, W1T, b1T, W2T, b2T):
    full = lambda a: pl.BlockSpec(a.shape, lambda b: (0,) * a.ndim)
    return pl.pallas_call(
        _k6_body,
        grid=(B,),
        in_specs=[pl.BlockSpec((1, 1, N, N), lambda b: (0, b, 0, 0)),
                  pl.BlockSpec((1, 1, N, N), lambda b: (1, b, 0, 0)),
                  pl.BlockSpec((1, N, H), lambda b: (b, 0, 0)),
                  full(W1T), full(b1T), full(W2T), full(b2T)],
        out_specs=pl.BlockSpec((1, 3, NA), lambda b: (b, 0, 0)),
        out_shape=jax.ShapeDtypeStruct((B, 3, NA), jnp.float32),
    )(A, A, h, W1T, b1T, W2T, b2T)


def kernel(x, edge_index, params):
    xs = x[:, :, 49:, :].transpose(0, 2, 1, 3)
    h0, s0, d0, P, Qt = _run_k1(xs, params)
    EW = _run_k2(P, Qt, params)
    src = edge_index[0]
    dst = edge_index[1]

    h, s, d = h0, s0, d0
    for l in range(3):
        parts = _run_sc(s.reshape(-1), d.reshape(-1), EW, src, dst)
        A = parts.reshape(2, B, N, N)
        if l < 2:
            nm = 'gat%d' % (l + 1)
            h, s, d = _run_k5(A, h, params[nm + '_W'],
                              params[nm + '_as'][0, 0], params[nm + '_ad'][0, 0])
        else:
            pred = _run_k6(A, h,
                           params['head_W1'].transpose(1, 2, 0),
                           params['head_b1'].transpose(1, 0),
                           params['head_W2'].transpose(1, 2, 0),
                           params['head_b2'].transpose(1, 0))
    return pred.transpose(0, 2, 1)


# speedup vs baseline: 169.9934x; 1.1650x over previous
import functools

import numpy as np
import jax
import jax.numpy as jnp
from jax import lax
from jax.experimental import pallas as pl
from jax.experimental.pallas import tpu as pltpu
from jax.experimental.pallas import tpu_sc as plsc

NA = 500
N = 512
H = 64
B = 4
RT = 15
EPR = N - 1
NE = N * EPR
EPW = 16 * EPR
BN_INV = np.float32(1.0 / np.sqrt(1.0 + 1e-5))
INV_SQRT2 = np.float32(0.7071067811865476)


def _gelu(v):
    return 0.5 * v * (1.0 + lax.erf(v * INV_SQRT2))


def _dg(a, b, a_dim, b_dim):
    return lax.dot_general(a, b, (((a_dim,), (b_dim,)), ((), ())),
                           preferred_element_type=jnp.float32)


def _k1_body(x_ref, embW_ref, embb_ref,
             c0W, c0b, c0g, c0be, c1W, c1b, c1g, c1be, c2W, c2b, c2g, c2be,
             gW, gas, gad, w1a_ref, w1b_ref, b1_ref,
             h0_ref, s0_ref, d0_ref, p_ref, q_ref):
    xb = x_ref[0]
    e = _dg(xb.reshape(RT * N, 32), embW_ref[...], 1, 1) + embb_ref[...][None, :]
    e = e.reshape(RT, N, H)

    def conv(hin, W_r, b_r, g_r, be_r, d, npos, inbase):
        acc = None
        for k in range(3):
            t0 = (RT - npos) - (2 - k) * d - inbase
            sl = hin[t0:t0 + npos]
            m = _dg(sl.reshape(npos * N, H), W_r[...][:, :, k], 1, 1)
            acc = m if acc is None else acc + m
        acc = acc + b_r[...][None, :]
        acc = g_r[...][None, :] * acc * BN_INV + be_r[...][None, :]
        return _gelu(acc).reshape(npos, N, H)

    l1 = conv(e, c0W, c0b, c0g, c0be, 1, 13, 0)
    l2 = conv(l1, c1W, c1b, c1g, c1be, 2, 9, 2)
    l3 = conv(l2, c2W, c2b, c2g, c2be, 4, 1, 6)
    feat = l3[0]

    h0 = _dg(feat, gW[...], 1, 1)
    h0_ref[...] = h0[None]
    s0_ref[...] = _dg(gas[...].reshape(1, H), h0, 1, 1)[None]
    d0_ref[...] = _dg(gad[...].reshape(1, H), h0, 1, 1)[None]
    p_ref[...] = (_dg(feat, w1a_ref[...], 1, 1) + b1_ref[...][None, :])[None]
    q_ref[...] = _dg(w1b_ref[...], feat, 1, 1)[None]


def _run_k1(xs, p):
    full = lambda a: pl.BlockSpec(a.shape, lambda b: (0,) * a.ndim)
    wargs = [p['emb_W'], p['emb_b'],
             p['conv0_W'], p['conv0_b'], p['conv0_g'], p['conv0_be'],
             p['conv1_W'], p['conv1_b'], p['conv1_g'], p['conv1_be'],
             p['conv2_W'], p['conv2_b'], p['conv2_g'], p['conv2_be'],
             p['gat0_W'], p['gat0_as'][0, 0], p['gat0_ad'][0, 0],
             p['ew_W1'][:, :H], p['ew_W1'][:, H:], p['ew_b1']]
    return pl.pallas_call(
        _k1_body,
        grid=(B,),
        in_specs=[pl.BlockSpec((1, RT, N, 32), lambda b: (b, 0, 0, 0))] +
                 [full(a) for a in wargs],
        out_specs=[pl.BlockSpec((1, N, H), lambda b: (b, 0, 0)),
                   pl.BlockSpec((1, 1, N), lambda b: (b, 0, 0)),
                   pl.BlockSpec((1, 1, N), lambda b: (b, 0, 0)),
                   pl.BlockSpec((1, N, H), lambda b: (b, 0, 0)),
                   pl.BlockSpec((1, H, N), lambda b: (b, 0, 0))],
        out_shape=[jax.ShapeDtypeStruct((B, N, H), jnp.float32),
                   jax.ShapeDtypeStruct((B, 1, N), jnp.float32),
                   jax.ShapeDtypeStruct((B, 1, N), jnp.float32),
                   jax.ShapeDtypeStruct((B, N, H), jnp.float32),
                   jax.ShapeDtypeStruct((B, H, N), jnp.float32)],
    )(xs, *wargs)


IB = 32


def _k2_body(p_ref, q_ref, w2_ref, b2_ref, ew_ref):
    pbT = jnp.transpose(p_ref[0], (1, 0))
    qt = q_ref[0]
    w2 = w2_ref[...]
    b2 = b2_ref[0, 0]
    for i in range(IB):
        t = jnp.maximum(pbT[:, i:i + 1] + qt, 0.0)
        z = jnp.sum(t * w2, axis=0, keepdims=True) + b2
        ew_ref[0, i:i + 1, :] = jax.nn.sigmoid(z)


def _run_k2(P, Q, p):
    w2 = p['ew_W2'].reshape(H, 1)
    b2 = p['ew_b2'].reshape(1, 1)
    return pl.pallas_call(
        _k2_body,
        grid=(B, N // IB),
        in_specs=[pl.BlockSpec((1, IB, H), lambda b, i: (b, i, 0)),
                  pl.BlockSpec((1, H, N), lambda b, i: (b, 0, 0)),
                  pl.BlockSpec((H, 1), lambda b, i: (0, 0)),
                  pl.BlockSpec((1, 1), lambda b, i: (0, 0))],
        out_specs=pl.BlockSpec((1, IB, N), lambda b, i: (b, i, 0)),
        out_shape=jax.ShapeDtypeStruct((B, N, N), jnp.float32),
    )(P, Q, w2, b2)


def _sc_body(s_hbm, d_hbm, ew_hbm, src_hbm, dst_hbm, out_hbm,
             src_c, dst_c, s_all, d_all, ew_r, p_buf, idx_buf, zbuf, apre,
             sem):
    cid = lax.axis_index("c")
    sid = lax.axis_index("s")
    wid = cid * 16 + sid

    pltpu.sync_copy(s_hbm, s_all)
    pltpu.sync_copy(d_hbm, d_all)
    pltpu.sync_copy(src_hbm.at[pl.ds(wid * EPW, EPW)], src_c.at[pl.ds(0, EPW)])
    pltpu.sync_copy(dst_hbm.at[pl.ds(wid * EPW, EPW)], dst_c.at[pl.ds(0, EPW)])
    src_c[pl.ds(EPW, 16)] = jnp.zeros((16,), jnp.int32)
    dst_c[pl.ds(EPW, 16)] = jnp.zeros((16,), jnp.int32)

    def _z(j, _):
        zbuf[pl.ds(j * 16, 16)] = jnp.zeros((16,), jnp.float32)
        return _
    lax.fori_loop(0, 128, _z, None)

    iota = lax.iota(jnp.int32, 16)
    HALF = 2 * N * N
    TSL = HALF // 16

    for ph in range(2):
        def _zf(k, _):
            pltpu.async_copy(zbuf, apre.at[pl.ds(sid * TSL + k * 2048, 2048)],
                             sem)
            return _
        lax.fori_loop(0, TSL // 2048, _zf, None)
        def _zd(k, _):
            pltpu.make_async_copy(
                zbuf, apre.at[pl.ds(sid * TSL + k * 2048, 2048)], sem).wait()
            return _
        lax.fori_loop(0, TSL // 2048, _zd, None)
        for b in range(2):
            pltpu.sync_copy(ew_hbm.at[2 * ph + b, pl.ds(wid * 16, 16)],
                            ew_r.at[pl.ds(b * 16, 16)])
        plsc.subcore_barrier()

        for i_loc in range(16):
            i_glob = wid * 16 + i_loc
            e_base = i_loc * EPR
            row_base = i_loc * 4

            @plsc.parallel_loop(0, 32, unroll=4)
            def step(t):
                j = t * 16 + iota
                valid = j < EPR
                jfull = jnp.where(valid, j + jnp.where(j >= i_glob, 1, 0), 0)
                e_loc = e_base + j
                srcv = plsc.load_gather(src_c, [e_loc])
                dstv = plsc.load_gather(dst_c, [e_loc])
                base = dstv * N + srcv
                row0 = row_base + t // 8
                col = (t % 8) * 16
                for b in range(2):
                    gb = 2 * ph + b
                    sg = plsc.load_gather(s_all, [gb * N + srcv])
                    dg = plsc.load_gather(d_all, [gb * N + dstv])
                    tt = sg + dg
                    r = jnp.where(tt > 0, tt, 0.2 * tt)
                    ewv = plsc.load_gather(
                        ew_r, [jnp.full((16,), b * 16 + i_loc, jnp.int32), jfull])
                    pv = jnp.where(valid, jnp.exp(r * ewv), 0.0)
                    p_buf[b * 64 + row0, pl.ds(col, 16)] = pv
                    idx_buf[b * 64 + row0, pl.ds(col, 16)] = base + b * (N * N)

        def scat_f(jr, _c):
            pltpu.async_copy(p_buf.at[jr], apre.at[idx_buf.at[jr]], sem,
                             add=True)
            return _c
        lax.fori_loop(0, 128, scat_f, None)
        def scat_d(jr, _c):
            pltpu.make_async_copy(p_buf.at[jr], apre.at[idx_buf.at[jr]],
                                  sem).wait()
            return _c
        lax.fori_loop(0, 128, scat_d, None)
        plsc.subcore_barrier()
        pltpu.sync_copy(
            apre.at[pl.ds(sid * TSL, TSL)],
            out_hbm.at[cid, pl.ds(ph * HALF + sid * TSL, TSL)])


def _run_sc(s, d, EW, src, dst):
    mesh = plsc.VectorSubcoreMesh(core_axis_name="c", subcore_axis_name="s")
    f = functools.partial(
        pl.kernel, _sc_body, mesh=mesh,
        compiler_params=pltpu.CompilerParams(needs_layout_passes=False),
        out_type=jax.ShapeDtypeStruct((2, B * N * N), jnp.float32),
        scratch_types=[
            pltpu.VMEM((EPW + 16,), jnp.int32),
            pltpu.VMEM((EPW + 16,), jnp.int32),
            pltpu.VMEM((B * N,), jnp.float32),
            pltpu.VMEM((B * N,), jnp.float32),
            pltpu.VMEM((32, N), jnp.float32),
            pltpu.VMEM((128, 128), jnp.float32),
            pltpu.VMEM((128, 128), jnp.int32),
            pltpu.VMEM((2048,), jnp.float32),
            pltpu.VMEM_SHARED((2 * N * N,), jnp.float32),
            pltpu.SemaphoreType.DMA,
        ],
    )()
    return f(s, d, EW, src, dst)


def _k5_body(a0_ref, a1_ref, h_ref, gW, gas, gad, h1_ref, s1_ref, d1_ref):
    A = a0_ref[0, 0] + a1_ref[0, 0]
    rs = jnp.sum(A, axis=1, keepdims=True)
    feat = _dg(A, h_ref[0], 1, 0) / (rs + 1e-8)
    h1 = _dg(feat, gW[...], 1, 1)
    h1_ref[...] = h1[None]
    s1_ref[...] = _dg(gas[...].reshape(1, H), h1, 1, 1)[None]
    d1_ref[...] = _dg(gad[...].reshape(1, H), h1, 1, 1)[None]


def _run_k5(A, h, W, a_s, a_d):
    full = lambda a: pl.BlockSpec(a.shape, lambda b: (0,) * a.ndim)
    return pl.pallas_call(
        _k5_body,
        grid=(B,),
        in_specs=[pl.BlockSpec((1, 1, N, N), lambda b: (0, b, 0, 0)),
                  pl.BlockSpec((1, 1, N, N), lambda b: (1, b, 0, 0)),
                  pl.BlockSpec((1, N, H), lambda b: (b, 0, 0)),
                  full(W), full(a_s), full(a_d)],
        out_specs=[pl.BlockSpec((1, N, H), lambda b: (b, 0, 0)),
                   pl.BlockSpec((1, 1, N), lambda b: (b, 0, 0)),
                   pl.BlockSpec((1, 1, N), lambda b: (b, 0, 0))],
        out_shape=[jax.ShapeDtypeStruct((B, N, H), jnp.float32),
                   jax.ShapeDtypeStruct((B, 1, N), jnp.float32),
                   jax.ShapeDtypeStruct((B, 1, N), jnp.float32)],
    )(A, A, h, W, a_s, a_d)


def _k6_body(a0_ref, a1_ref, h_ref, w1_ref, b1_ref, w2_ref, b2_ref, out_ref):
    A = a0_ref[0, 0] + a1_ref[0, 0]
    rs = jnp.sum(A, axis=1, keepdims=True)
    feat = _dg(A, h_ref[0], 1, 0) / (rs + 1e-8)
    afT = jnp.transpose(feat[:NA], (1, 0))
    t1 = jnp.concatenate(
        [jnp.sum(afT * w1_ref[o], axis=0, keepdims=True) for o in range(32)],
        axis=0) + b1_ref[...]
    t1 = jnp.maximum(t1, 0.0)
    pred = jnp.concatenate(
        [jnp.sum(t1 * w2_ref[o], axis=0, keepdims=True) for o in range(3)],
        axis=0) + b2_ref[...]
    out_ref[...] = pred[None]


def _run_k6(A, h, W1T, b1T, W2T, b2T):
    full = lambda a: pl.BlockSpec(a.shape, lambda b: (0,) * a.ndim)
    return pl.pallas_call(
        _k6_body,
        grid=(B,),
        in_specs=[pl.BlockSpec((1, 1, N, N), lambda b: (0, b, 0, 0)),
                  pl.BlockSpec((1, 1, N, N), lambda b: (1, b, 0, 0)),
                  pl.BlockSpec((1, N, H), lambda b: (b, 0, 0)),
                  full(W1T), full(b1T), full(W2T), full(b2T)],
        out_specs=pl.BlockSpec((1, 3, NA), lambda b: (b, 0, 0)),
        out_shape=jax.ShapeDtypeStruct((B, 3, NA), jnp.float32),
    )(A, A, h, W1T, b1T, W2T, b2T)


def kernel(x, edge_index, params):
    xs = x[:, :, 49:, :].transpose(0, 2, 1, 3)
    h0, s0, d0, P, Qt = _run_k1(xs, params)
    EW = _run_k2(P, Qt, params)
    src = edge_index[0]
    dst = edge_index[1]

    h, s, d = h0, s0, d0
    for l in range(3):
        parts = _run_sc(s.reshape(-1), d.reshape(-1), EW, src, dst)
        A = parts.reshape(2, B, N, N)
        if l < 2:
            nm = 'gat%d' % (l + 1)
            h, s, d = _run_k5(A, h, params[nm + '_W'],
                              params[nm + '_as'][0, 0], params[nm + '_ad'][0, 0])
        else:
            pred = _run_k6(A, h,
                           params['head_W1'].transpose(1, 2, 0),
                           params['head_b1'].transpose(1, 0),
                           params['head_W2'].transpose(1, 2, 0),
                           params['head_b2'].transpose(1, 0))
    return pred.transpose(0, 2, 1)
